# Initial kernel scaffold; baseline (speedup 1.0000x reference)
import functools

import jax
import jax.numpy as jnp
from jax import lax
from jax.experimental import pallas as pl
from jax.experimental.pallas import tpu as pltpu
from jax.experimental.pallas import tpu_sc as plsc

N = 10000
E = 320000
FEATURES = 128
HIDDEN = 64
CLASSES = 16
LAYER_NUM = 4
EPS = 0.3

ROWW = 80
DROWW = 16
NC = 2
NS = 16
NW = NC * NS
EPW = E // NW
CHUNK = 80
NCHUNK = EPW // CHUNK
STRIPE = N // NS


def _fill_tables(x, Wgd_ref, Wgs_ref, bg_ref, deg_ref, st_ref, dt_ref):
    a_s = jnp.dot(x, Wgs_ref[...], preferred_element_type=jnp.float32)
    a_d = jnp.dot(x, Wgd_ref[...], preferred_element_type=jnp.float32)
    a_d = a_d + bg_ref[...]
    deg = deg_ref[...]
    pad_s = jnp.zeros((x.shape[0], ROWW - HIDDEN - 2), jnp.float32)
    st_ref[...] = jnp.concatenate([x, a_s, deg, pad_s], axis=1)
    pad_d = jnp.zeros((x.shape[0], DROWW - 2), jnp.float32)
    dt_ref[...] = jnp.concatenate([a_d, deg, pad_d], axis=1)


def _encode_body(h_ref, W1_ref, b1_ref, Wgd_ref, Wgs_ref, bg_ref, deg_ref,
                 st_ref, dt_ref, x_ref):
    x = jnp.dot(h_ref[...], W1_ref[...], preferred_element_type=jnp.float32)
    x = jnp.maximum(x + b1_ref[...], 0.0)
    _fill_tables(x, Wgd_ref, Wgs_ref, bg_ref, deg_ref, st_ref, dt_ref)
    x_ref[...] = x


def _update_body(p_ref, h0_ref, deg_ref, Wgd_ref, Wgs_ref, bg_ref,
                 st_ref, dt_ref):
    x = EPS * h0_ref[...] + p_ref[0] + p_ref[1]
    _fill_tables(x, Wgd_ref, Wgs_ref, bg_ref, deg_ref, st_ref, dt_ref)


def _final_body(p_ref, h0_ref, W2_ref, b2_ref, o_ref):
    x = EPS * h0_ref[...] + p_ref[0] + p_ref[1]
    logits = jnp.dot(x, W2_ref[...], preferred_element_type=jnp.float32)
    logits = logits + b2_ref[...]
    m = jnp.max(logits, axis=1, keepdims=True)
    ex = jnp.exp(logits - m)
    o_ref[...] = logits - m - jnp.log(jnp.sum(ex, axis=1, keepdims=True))


def _encode(h, W1, b1, Wgd, Wgs, bgl, deg):
    return pl.pallas_call(
        _encode_body,
        out_shape=(
            jax.ShapeDtypeStruct((N, ROWW), jnp.float32),
            jax.ShapeDtypeStruct((N, DROWW), jnp.float32),
            jax.ShapeDtypeStruct((N, HIDDEN), jnp.float32),
        ),
    )(h, W1, b1, Wgd, Wgs, bgl, deg)


def _update(parts, h0, deg, Wgd, Wgs, bgl):
    return pl.pallas_call(
        _update_body,
        out_shape=(
            jax.ShapeDtypeStruct((N, ROWW), jnp.float32),
            jax.ShapeDtypeStruct((N, DROWW), jnp.float32),
        ),
    )(parts, h0, deg, Wgd, Wgs, bgl)


def _final(parts, h0, W2, b2):
    return pl.pallas_call(
        _final_body,
        out_shape=jax.ShapeDtypeStruct((N, CLASSES), jnp.float32),
    )(parts, h0, W2, b2)


_MESH = plsc.VectorSubcoreMesh(core_axis_name="c", subcore_axis_name="s")


@functools.partial(
    pl.kernel,
    out_type=jax.ShapeDtypeStruct((NC, N, HIDDEN), jnp.float32),
    mesh=_MESH,
    scratch_types=[
        pltpu.VMEM((CHUNK,), jnp.int32),
        pltpu.VMEM((CHUNK,), jnp.int32),
        pltpu.VMEM((CHUNK, ROWW), jnp.float32),
        pltpu.VMEM((CHUNK, DROWW), jnp.float32),
        pltpu.VMEM((CHUNK, HIDDEN), jnp.float32),
        pltpu.VMEM((CHUNK,), jnp.float32),
        pltpu.VMEM_SHARED((N, HIDDEN), jnp.float32),
    ],
)
def _sc_aggregate(st_hbm, dt_hbm, src_hbm, dst_hbm, zeros_hbm, out_hbm,
                  sidx, didx, srows, drows, scaled, evec, agg):
    cid = lax.axis_index("c")
    sid = lax.axis_index("s")
    wid = sid * NC + cid

    pltpu.sync_copy(zeros_hbm, agg.at[pl.ds(sid * STRIPE, STRIPE)])
    plsc.subcore_barrier()

    base = wid * EPW

    @pl.loop(0, NCHUNK)
    def _chunk(c):
        off = base + c * CHUNK
        pltpu.sync_copy(src_hbm.at[pl.ds(off, CHUNK)], sidx)
        pltpu.sync_copy(dst_hbm.at[pl.ds(off, CHUNK)], didx)
        pltpu.sync_copy(st_hbm.at[sidx], srows)
        pltpu.sync_copy(dt_hbm.at[didx], drows)

        @pl.loop(0, CHUNK, step=16)
        def _gate(i):
            rows = lax.iota(jnp.int32, 16) + i
            c0 = jnp.zeros((16,), jnp.int32)
            a_d = plsc.load_gather(drows, [rows, c0])
            d_d = plsc.load_gather(drows, [rows, c0 + 1])
            a_s = plsc.load_gather(srows, [rows, c0 + HIDDEN])
            d_s = plsc.load_gather(srows, [rows, c0 + HIDDEN + 1])
            z = a_d + a_s
            t = 1.0 - 2.0 / (jnp.exp(2.0 * z) + 1.0)
            evec[pl.ds(i, 16)] = t * d_d * d_s

        @pl.loop(0, CHUNK)
        def _scale(i):
            e = evec[i]
            for j in range(HIDDEN // 16):
                scaled[i, pl.ds(j * 16, 16)] = srows[i, pl.ds(j * 16, 16)] * e

        pltpu.sync_copy(scaled, agg.at[didx], add=True)

    plsc.subcore_barrier()
    pltpu.sync_copy(agg.at[pl.ds(sid * STRIPE, STRIPE)],
                    out_hbm.at[cid, pl.ds(sid * STRIPE, STRIPE)])


def kernel(h, adj, deg, W1, b1, Wg, bg, W2, b2):
    h = h.astype(jnp.float32)
    src = adj[0].astype(jnp.int32)
    dst = adj[1].astype(jnp.int32)
    deg2 = deg.astype(jnp.float32).reshape(N, 1)
    b1r = b1.astype(jnp.float32).reshape(1, HIDDEN)
    b2r = b2.astype(jnp.float32).reshape(1, CLASSES)
    Wg = Wg.astype(jnp.float32)
    bgr = bg.astype(jnp.float32).reshape(LAYER_NUM, 1, 1)
    zeros = jnp.zeros((STRIPE, HIDDEN), jnp.float32)

    st, dt, h0 = _encode(h, W1.astype(jnp.float32), b1r,
                         Wg[0, :HIDDEN], Wg[0, HIDDEN:], bgr[0], deg2)
    parts = None
    for l in range(LAYER_NUM):
        parts = _sc_aggregate(st, dt, src, dst, zeros)
        if l + 1 < LAYER_NUM:
            st, dt = _update(parts, h0, deg2,
                             Wg[l + 1, :HIDDEN], Wg[l + 1, HIDDEN:],
                             bgr[l + 1])
    return _final(parts, h0, W2.astype(jnp.float32), b2r)


# speedup vs baseline: 93.6641x; 93.6641x over previous
import dataclasses
import functools

import jax
import jax.numpy as jnp
from jax import lax
from jax.experimental import pallas as pl
from jax.experimental.pallas import tpu as pltpu
from jax.experimental.pallas import tpu_sc as plsc

N = 10000
E = 320000
FEATURES = 128
HIDDEN = 64
CLASSES = 16
LAYER_NUM = 4
EPS = 0.3

ROWW = 80
DROWW = 16
NC = 2
NS = 16
NW = NC * NS
EPW = E // NW
CHUNK = 80
NCHUNK = EPW // CHUNK
NPAD = 10240
STRIPE = NPAD // NS


def _fill_tables(x, Wgd_ref, Wgs_ref, bg_ref, deg_ref, st_ref, dt_ref):
    a_s = jnp.dot(x, Wgs_ref[...], preferred_element_type=jnp.float32)
    a_d = jnp.dot(x, Wgd_ref[...], preferred_element_type=jnp.float32)
    a_d = a_d + bg_ref[...]
    deg = deg_ref[...]
    pad_s = jnp.zeros((x.shape[0], ROWW - HIDDEN - 2), jnp.float32)
    st_ref[...] = jnp.concatenate([x, a_s, deg, pad_s], axis=1)
    pad_d = jnp.zeros((x.shape[0], DROWW - 2), jnp.float32)
    dt_ref[...] = jnp.concatenate([a_d, deg, pad_d], axis=1)


def _encode_body(h_ref, W1_ref, b1_ref, Wgd_ref, Wgs_ref, bg_ref, deg_ref,
                 st_ref, dt_ref, x_ref):
    x = jnp.dot(h_ref[...], W1_ref[...], preferred_element_type=jnp.float32)
    x = jnp.maximum(x + b1_ref[...], 0.0)
    _fill_tables(x, Wgd_ref, Wgs_ref, bg_ref, deg_ref, st_ref, dt_ref)
    x_ref[...] = x


def _update_body(p_ref, h0_ref, deg_ref, Wgd_ref, Wgs_ref, bg_ref,
                 st_ref, dt_ref):
    x = EPS * h0_ref[...] + p_ref[0, :N] + p_ref[1, :N]
    _fill_tables(x, Wgd_ref, Wgs_ref, bg_ref, deg_ref, st_ref, dt_ref)


def _final_body(p_ref, h0_ref, W2_ref, b2_ref, o_ref):
    x = EPS * h0_ref[...] + p_ref[0, :N] + p_ref[1, :N]
    logits = jnp.dot(x, W2_ref[...], preferred_element_type=jnp.float32)
    logits = logits + b2_ref[...]
    m = jnp.max(logits, axis=1, keepdims=True)
    ex = jnp.exp(logits - m)
    o_ref[...] = logits - m - jnp.log(jnp.sum(ex, axis=1, keepdims=True))


def _encode(h, W1, b1, Wgd, Wgs, bgl, deg):
    return pl.pallas_call(
        _encode_body,
        out_shape=(
            jax.ShapeDtypeStruct((N, ROWW), jnp.float32),
            jax.ShapeDtypeStruct((N, DROWW), jnp.float32),
            jax.ShapeDtypeStruct((N, HIDDEN), jnp.float32),
        ),
    )(h, W1, b1, Wgd, Wgs, bgl, deg)


def _update(parts, h0, deg, Wgd, Wgs, bgl):
    return pl.pallas_call(
        _update_body,
        out_shape=(
            jax.ShapeDtypeStruct((N, ROWW), jnp.float32),
            jax.ShapeDtypeStruct((N, DROWW), jnp.float32),
        ),
    )(parts, h0, deg, Wgd, Wgs, bgl)


def _final(parts, h0, W2, b2):
    return pl.pallas_call(
        _final_body,
        out_shape=jax.ShapeDtypeStruct((N, CLASSES), jnp.float32),
    )(parts, h0, W2, b2)


_MESH = plsc.VectorSubcoreMesh(core_axis_name="c", subcore_axis_name="s")

_SC_PARAMS = pltpu.CompilerParams(use_tc_tiling_on_sc=False)
if "needs_layout_passes" in pltpu.CompilerParams.__dataclass_fields__:
    _SC_PARAMS = dataclasses.replace(_SC_PARAMS, needs_layout_passes=False)


@functools.partial(
    pl.kernel,
    out_type=jax.ShapeDtypeStruct((NC, NPAD, HIDDEN), jnp.float32),
    mesh=_MESH,
    compiler_params=_SC_PARAMS,
    scratch_types=[
        pltpu.VMEM((CHUNK,), jnp.int32),
        pltpu.VMEM((CHUNK,), jnp.int32),
        pltpu.VMEM((CHUNK, ROWW), jnp.float32),
        pltpu.VMEM((CHUNK, DROWW), jnp.float32),
        pltpu.VMEM((CHUNK, HIDDEN), jnp.float32),
        pltpu.VMEM((CHUNK,), jnp.float32),
        pltpu.VMEM_SHARED((NPAD, HIDDEN), jnp.float32),
    ],
)
def _sc_aggregate(st_hbm, dt_hbm, src_hbm, dst_hbm, zeros_hbm, out_hbm,
                  sidx, didx, srows, drows, scaled, evec, agg):
    cid = lax.axis_index("c")
    sid = lax.axis_index("s")
    wid = sid * jnp.int32(NC) + cid
    srow0 = sid * jnp.int32(STRIPE)

    pltpu.sync_copy(zeros_hbm, agg.at[pl.ds(srow0, STRIPE)])
    plsc.subcore_barrier()

    base = wid * jnp.int32(EPW)

    def _chunk(c, carry):
        off = base + c * jnp.int32(CHUNK)
        pltpu.sync_copy(src_hbm.at[pl.ds(off, CHUNK)], sidx)
        pltpu.sync_copy(dst_hbm.at[pl.ds(off, CHUNK)], didx)
        pltpu.sync_copy(st_hbm.at[sidx], srows)
        pltpu.sync_copy(dt_hbm.at[didx], drows)

        def _gate(g, carry2):
            i = g * jnp.int32(16)
            rows = lax.iota(jnp.int32, 16) + i
            c0 = jnp.zeros((16,), jnp.int32)
            a_d = plsc.load_gather(drows, [rows, c0])
            d_d = plsc.load_gather(drows, [rows, c0 + 1])
            a_s = plsc.load_gather(srows, [rows, c0 + HIDDEN])
            d_s = plsc.load_gather(srows, [rows, c0 + HIDDEN + 1])
            z = a_d + a_s
            t = 1.0 - 2.0 / (jnp.exp(2.0 * z) + 1.0)
            evec[pl.ds(i, 16)] = t * d_d * d_s
            return carry2

        lax.fori_loop(jnp.int32(0), jnp.int32(CHUNK // 16), _gate, 0,
                      unroll=False)

        def _scale(i, carry2):
            e = plsc.load_gather(evec, [jnp.full((16,), i, jnp.int32)])
            for j in range(HIDDEN // 16):
                scaled[i, pl.ds(j * 16, 16)] = srows[i, pl.ds(j * 16, 16)] * e
            return carry2

        lax.fori_loop(jnp.int32(0), jnp.int32(CHUNK), _scale, 0,
                      unroll=False)

        pltpu.sync_copy(scaled, agg.at[didx], add=True)
        return carry

    lax.fori_loop(jnp.int32(0), jnp.int32(NCHUNK), _chunk, 0, unroll=False)

    plsc.subcore_barrier()
    pltpu.sync_copy(agg.at[pl.ds(srow0, STRIPE)],
                    out_hbm.at[cid, pl.ds(srow0, STRIPE)])


def kernel(h, adj, deg, W1, b1, Wg, bg, W2, b2):
    h = h.astype(jnp.float32)
    src = adj[0].astype(jnp.int32)
    dst = adj[1].astype(jnp.int32)
    deg2 = deg.astype(jnp.float32).reshape(N, 1)
    b1r = b1.astype(jnp.float32).reshape(1, HIDDEN)
    b2r = b2.astype(jnp.float32).reshape(1, CLASSES)
    Wg = Wg.astype(jnp.float32)
    bgr = bg.astype(jnp.float32).reshape(LAYER_NUM, 1, 1)
    zeros = jnp.zeros((STRIPE, HIDDEN), jnp.float32)

    st, dt, h0 = _encode(h, W1.astype(jnp.float32), b1r,
                         Wg[0, :HIDDEN], Wg[0, HIDDEN:], bgr[0], deg2)
    parts = None
    for l in range(LAYER_NUM):
        parts = _sc_aggregate(st, dt, src, dst, zeros)
        if l + 1 < LAYER_NUM:
            st, dt = _update(parts, h0, deg2,
                             Wg[l + 1, :HIDDEN], Wg[l + 1, HIDDEN:],
                             bgr[l + 1])
    out = _final(parts, h0, W2.astype(jnp.float32), b2r)
    return out.astype(jnp.float64)


# speedup vs baseline: 252.7805x; 2.6988x over previous
import dataclasses
import functools

import jax
import jax.numpy as jnp
from jax import lax
from jax.experimental import pallas as pl
from jax.experimental.pallas import tpu as pltpu
from jax.experimental.pallas import tpu_sc as plsc

N = 10000
E = 320000
FEATURES = 128
HIDDEN = 64
CLASSES = 16
LAYER_NUM = 4
EPS = 0.3

ROWW = 80
DROWW = 16
NC = 2
NS = 16
NW = NC * NS
EPW = E // NW
CHUNK = 80
NCHUNK = EPW // CHUNK
NPAD = 10240
STRIPE = NPAD // NS


def _fill_tables(x, Wgd_ref, Wgs_ref, bg_ref, deg_ref, st_ref, dt_ref):
    a_s = jnp.dot(x, Wgs_ref[...], preferred_element_type=jnp.float32)
    a_d = jnp.dot(x, Wgd_ref[...], preferred_element_type=jnp.float32)
    a_d = a_d + bg_ref[...]
    deg = deg_ref[...]
    pad_s = jnp.zeros((x.shape[0], ROWW - HIDDEN - 2), jnp.float32)
    st_ref[...] = jnp.concatenate([x, a_s, deg, pad_s], axis=1)
    pad_d = jnp.zeros((x.shape[0], DROWW - 2), jnp.float32)
    dt_ref[...] = jnp.concatenate([a_d, deg, pad_d], axis=1)


def _encode_body(h_ref, W1_ref, b1_ref, Wgd_ref, Wgs_ref, bg_ref, deg_ref,
                 st_ref, dt_ref, x_ref):
    x = jnp.dot(h_ref[...], W1_ref[...], preferred_element_type=jnp.float32)
    x = jnp.maximum(x + b1_ref[...], 0.0)
    _fill_tables(x, Wgd_ref, Wgs_ref, bg_ref, deg_ref, st_ref, dt_ref)
    x_ref[...] = x


def _update_body(p_ref, h0_ref, deg_ref, Wgd_ref, Wgs_ref, bg_ref,
                 st_ref, dt_ref):
    x = EPS * h0_ref[...] + p_ref[0, :N] + p_ref[1, :N]
    _fill_tables(x, Wgd_ref, Wgs_ref, bg_ref, deg_ref, st_ref, dt_ref)


def _final_body(p_ref, h0_ref, W2_ref, b2_ref, o_ref):
    x = EPS * h0_ref[...] + p_ref[0, :N] + p_ref[1, :N]
    logits = jnp.dot(x, W2_ref[...], preferred_element_type=jnp.float32)
    logits = logits + b2_ref[...]
    m = jnp.max(logits, axis=1, keepdims=True)
    ex = jnp.exp(logits - m)
    o_ref[...] = logits - m - jnp.log(jnp.sum(ex, axis=1, keepdims=True))


def _encode(h, W1, b1, Wgd, Wgs, bgl, deg):
    return pl.pallas_call(
        _encode_body,
        out_shape=(
            jax.ShapeDtypeStruct((N, ROWW), jnp.float32),
            jax.ShapeDtypeStruct((N, DROWW), jnp.float32),
            jax.ShapeDtypeStruct((N, HIDDEN), jnp.float32),
        ),
    )(h, W1, b1, Wgd, Wgs, bgl, deg)


def _update(parts, h0, deg, Wgd, Wgs, bgl):
    return pl.pallas_call(
        _update_body,
        out_shape=(
            jax.ShapeDtypeStruct((N, ROWW), jnp.float32),
            jax.ShapeDtypeStruct((N, DROWW), jnp.float32),
        ),
    )(parts, h0, deg, Wgd, Wgs, bgl)


def _final(parts, h0, W2, b2):
    return pl.pallas_call(
        _final_body,
        out_shape=jax.ShapeDtypeStruct((N, CLASSES), jnp.float32),
    )(parts, h0, W2, b2)


_MESH = plsc.VectorSubcoreMesh(core_axis_name="c", subcore_axis_name="s")

_SC_PARAMS = pltpu.CompilerParams(use_tc_tiling_on_sc=False)
if "needs_layout_passes" in pltpu.CompilerParams.__dataclass_fields__:
    _SC_PARAMS = dataclasses.replace(_SC_PARAMS, needs_layout_passes=False)


@functools.partial(
    pl.kernel,
    out_type=jax.ShapeDtypeStruct((NC, NPAD, HIDDEN), jnp.float32),
    mesh=_MESH,
    compiler_params=_SC_PARAMS,
    scratch_types=[
        pltpu.VMEM((NCHUNK, CHUNK), jnp.int32),
        pltpu.VMEM((NCHUNK, CHUNK), jnp.int32),
        pltpu.VMEM((CHUNK, ROWW), jnp.float32),
        pltpu.VMEM((CHUNK, ROWW), jnp.float32),
        pltpu.VMEM((CHUNK, DROWW), jnp.float32),
        pltpu.VMEM((CHUNK, DROWW), jnp.float32),
        pltpu.VMEM((CHUNK, HIDDEN), jnp.float32),
        pltpu.VMEM((CHUNK, HIDDEN), jnp.float32),
        pltpu.VMEM((CHUNK,), jnp.float32),
        pltpu.VMEM_SHARED((NPAD, HIDDEN), jnp.float32),
        pltpu.SemaphoreType.DMA,
        pltpu.SemaphoreType.DMA,
        pltpu.SemaphoreType.DMA,
        pltpu.SemaphoreType.DMA,
        pltpu.SemaphoreType.DMA,
        pltpu.SemaphoreType.DMA,
    ],
)
def _sc_aggregate(st_hbm, dt_hbm, src_hbm, dst_hbm, zeros_hbm, out_hbm,
                  sidx, didx, srA, srB, drA, drB, scA, scB, evec, agg,
                  sem_sA, sem_sB, sem_dA, sem_dB, sem_cA, sem_cB):
    cid = lax.axis_index("c")
    sid = lax.axis_index("s")
    wid = sid * jnp.int32(NC) + cid
    srow0 = sid * jnp.int32(STRIPE)

    pltpu.sync_copy(zeros_hbm, agg.at[pl.ds(srow0, STRIPE)])
    plsc.subcore_barrier()

    pltpu.sync_copy(src_hbm.at[wid], sidx)
    pltpu.sync_copy(dst_hbm.at[wid], didx)

    def start_gather(c, sr, dr, sem_s, sem_d):
        pltpu.async_copy(st_hbm.at[sidx.at[c]], sr, sem_s)
        pltpu.async_copy(dt_hbm.at[didx.at[c]], dr, sem_d)

    def process(c, sr, dr, sc, sem_s, sem_d, sem_c, first):
        pltpu.make_async_copy(st_hbm.at[sidx.at[c]], sr, sem_s).wait()
        pltpu.make_async_copy(dt_hbm.at[didx.at[c]], dr, sem_d).wait()
        for g in range(CHUNK // 16):
            rows = lax.iota(jnp.int32, 16) + jnp.int32(16 * g)
            c0 = jnp.zeros((16,), jnp.int32)
            a_d = plsc.load_gather(dr, [rows, c0])
            d_d = plsc.load_gather(dr, [rows, c0 + 1])
            a_s = plsc.load_gather(sr, [rows, c0 + HIDDEN])
            d_s = plsc.load_gather(sr, [rows, c0 + HIDDEN + 1])
            t = 1.0 - 2.0 / (jnp.exp(2.0 * (a_d + a_s)) + 1.0)
            evec[pl.ds(16 * g, 16)] = t * d_d * d_s
        @pl.when(jnp.logical_not(first))
        def _():
            pltpu.make_async_copy(sc, agg.at[didx.at[c]], sem_c).wait()
        for i in range(CHUNK):
            e = plsc.load_gather(evec, [jnp.full((16,), i, jnp.int32)])
            for j in range(HIDDEN // 16):
                sc[i, pl.ds(j * 16, 16)] = sr[i, pl.ds(j * 16, 16)] * e
        pltpu.async_copy(sc, agg.at[didx.at[c]], sem_c, add=True)

    start_gather(jnp.int32(0), srA, drA, sem_sA, sem_dA)

    def _pair(p, carry):
        c0 = p * jnp.int32(2)
        start_gather(c0 + 1, srB, drB, sem_sB, sem_dB)
        process(c0, srA, drA, scA, sem_sA, sem_dA, sem_cA, p == 0)
        start_gather(c0 + 2, srA, drA, sem_sA, sem_dA)
        process(c0 + 1, srB, drB, scB, sem_sB, sem_dB, sem_cB, p == 0)
        return carry

    lax.fori_loop(jnp.int32(0), jnp.int32((NCHUNK - 1) // 2), _pair, 0,
                  unroll=False)

    process(jnp.int32(NCHUNK - 1), srA, drA, scA, sem_sA, sem_dA, sem_cA,
            jnp.bool_(False))

    pltpu.make_async_copy(scB, agg.at[didx.at[jnp.int32(0)]], sem_cB).wait()
    pltpu.make_async_copy(scA, agg.at[didx.at[jnp.int32(0)]], sem_cA).wait()

    plsc.subcore_barrier()
    pltpu.sync_copy(agg.at[pl.ds(srow0, STRIPE)],
                    out_hbm.at[cid, pl.ds(srow0, STRIPE)])


def kernel(h, adj, deg, W1, b1, Wg, bg, W2, b2):
    h = h.astype(jnp.float32)
    src = adj[0].astype(jnp.int32).reshape(NW, NCHUNK, CHUNK)
    dst = adj[1].astype(jnp.int32).reshape(NW, NCHUNK, CHUNK)
    deg2 = deg.astype(jnp.float32).reshape(N, 1)
    b1r = b1.astype(jnp.float32).reshape(1, HIDDEN)
    b2r = b2.astype(jnp.float32).reshape(1, CLASSES)
    Wg = Wg.astype(jnp.float32)
    bgr = bg.astype(jnp.float32).reshape(LAYER_NUM, 1, 1)
    zeros = jnp.zeros((STRIPE, HIDDEN), jnp.float32)

    st, dt, h0 = _encode(h, W1.astype(jnp.float32), b1r,
                         Wg[0, :HIDDEN], Wg[0, HIDDEN:], bgr[0], deg2)
    parts = None
    for l in range(LAYER_NUM):
        parts = _sc_aggregate(st, dt, src, dst, zeros)
        if l + 1 < LAYER_NUM:
            st, dt = _update(parts, h0, deg2,
                             Wg[l + 1, :HIDDEN], Wg[l + 1, HIDDEN:],
                             bgr[l + 1])
    out = _final(parts, h0, W2.astype(jnp.float32), b2r)
    return out.astype(jnp.float64)


# speedup vs baseline: 371.9116x; 1.4713x over previous
import dataclasses
import functools

import jax
import jax.numpy as jnp
import numpy as np
from jax import lax
from jax.experimental import pallas as pl
from jax.experimental.pallas import tpu as pltpu
from jax.experimental.pallas import tpu_sc as plsc

N = 10000
E = 320000
FEATURES = 128
HIDDEN = 64
CLASSES = 16
LAYER_NUM = 4
EPS = 0.3

ROWW = 80
DROWW = 16
NC = 2
NS = 16
NW = NC * NS
EPW = E // NW
CHUNK = 80
NCHUNK = EPW // CHUNK
NPAD = 10240
STRIPE = NPAD // NS


def _fill_tables(x, Wgd_ref, Wgs_ref, bg_ref, deg_ref, st_ref, dt_ref):
    a_s = jnp.dot(x, Wgs_ref[...], preferred_element_type=jnp.float32)
    a_d = jnp.dot(x, Wgd_ref[...], preferred_element_type=jnp.float32)
    a_d = a_d + bg_ref[...]
    deg = deg_ref[...]
    pad_s = jnp.zeros((x.shape[0], ROWW - HIDDEN - 2), jnp.float32)
    st_ref[...] = jnp.concatenate([x, a_s, deg, pad_s], axis=1)
    pad_d = jnp.zeros((x.shape[0], DROWW - 2), jnp.float32)
    dt_ref[...] = jnp.concatenate([a_d, deg, pad_d], axis=1)


def _encode_body(h_ref, W1_ref, b1_ref, Wgd_ref, Wgs_ref, bg_ref, deg_ref,
                 st_ref, dt_ref, x_ref):
    x = jnp.dot(h_ref[...], W1_ref[...], preferred_element_type=jnp.float32)
    x = jnp.maximum(x + b1_ref[...], 0.0)
    _fill_tables(x, Wgd_ref, Wgs_ref, bg_ref, deg_ref, st_ref, dt_ref)
    x_ref[...] = x


def _update_body(p_ref, h0_ref, deg_ref, Wgd_ref, Wgs_ref, bg_ref,
                 st_ref, dt_ref):
    x = EPS * h0_ref[...] + p_ref[0, :N] + p_ref[1, :N]
    _fill_tables(x, Wgd_ref, Wgs_ref, bg_ref, deg_ref, st_ref, dt_ref)


def _final_body(p_ref, h0_ref, W2_ref, b2_ref, o_ref):
    x = EPS * h0_ref[...] + p_ref[0, :N] + p_ref[1, :N]
    logits = jnp.dot(x, W2_ref[...], preferred_element_type=jnp.float32)
    logits = logits + b2_ref[...]
    m = jnp.max(logits, axis=1, keepdims=True)
    ex = jnp.exp(logits - m)
    o_ref[...] = logits - m - jnp.log(jnp.sum(ex, axis=1, keepdims=True))


def _encode(h, W1, b1, Wgd, Wgs, bgl, deg):
    return pl.pallas_call(
        _encode_body,
        out_shape=(
            jax.ShapeDtypeStruct((N, ROWW), jnp.float32),
            jax.ShapeDtypeStruct((N, DROWW), jnp.float32),
            jax.ShapeDtypeStruct((N, HIDDEN), jnp.float32),
        ),
    )(h, W1, b1, Wgd, Wgs, bgl, deg)


def _update(parts, h0, deg, Wgd, Wgs, bgl):
    return pl.pallas_call(
        _update_body,
        out_shape=(
            jax.ShapeDtypeStruct((N, ROWW), jnp.float32),
            jax.ShapeDtypeStruct((N, DROWW), jnp.float32),
        ),
    )(parts, h0, deg, Wgd, Wgs, bgl)


def _final(parts, h0, W2, b2):
    return pl.pallas_call(
        _final_body,
        out_shape=jax.ShapeDtypeStruct((N, CLASSES), jnp.float32),
    )(parts, h0, W2, b2)


_MESH = plsc.VectorSubcoreMesh(core_axis_name="c", subcore_axis_name="s")

_SC_PARAMS = pltpu.CompilerParams(use_tc_tiling_on_sc=False)
if "needs_layout_passes" in pltpu.CompilerParams.__dataclass_fields__:
    _SC_PARAMS = dataclasses.replace(_SC_PARAMS, needs_layout_passes=False)


@functools.partial(
    pl.kernel,
    out_type=jax.ShapeDtypeStruct((NC, NPAD, HIDDEN), jnp.float32),
    mesh=_MESH,
    compiler_params=_SC_PARAMS,
    scratch_types=[
        pltpu.VMEM((NCHUNK, CHUNK), jnp.int32),
        pltpu.VMEM((NCHUNK, CHUNK), jnp.int32),
        pltpu.VMEM((CHUNK, ROWW), jnp.float32),
        pltpu.VMEM((CHUNK, ROWW), jnp.float32),
        pltpu.VMEM((CHUNK, DROWW), jnp.float32),
        pltpu.VMEM((CHUNK, DROWW), jnp.float32),
        pltpu.VMEM((CHUNK, HIDDEN), jnp.float32),
        pltpu.VMEM((CHUNK, HIDDEN), jnp.float32),
        pltpu.VMEM((CHUNK,), jnp.float32),
        pltpu.VMEM_SHARED((NPAD, HIDDEN), jnp.float32),
        pltpu.SemaphoreType.DMA,
        pltpu.SemaphoreType.DMA,
        pltpu.SemaphoreType.DMA,
        pltpu.SemaphoreType.DMA,
        pltpu.SemaphoreType.DMA,
        pltpu.SemaphoreType.DMA,
    ],
)
def _sc_aggregate(st_hbm, dt_hbm, src_hbm, dst_hbm, zeros_hbm, out_hbm,
                  sidx, didx, srA, srB, drA, drB, scA, scB, evec, agg,
                  sem_sA, sem_sB, sem_dA, sem_dB, sem_cA, sem_cB):
    cid = lax.axis_index("c")
    sid = lax.axis_index("s")
    wid = sid * jnp.int32(NC) + cid
    srow0 = sid * jnp.int32(STRIPE)

    pltpu.sync_copy(zeros_hbm, agg.at[pl.ds(srow0, STRIPE)])
    plsc.subcore_barrier()

    pltpu.sync_copy(src_hbm.at[wid], sidx)
    pltpu.sync_copy(dst_hbm.at[wid], didx)

    def start_gather(c, sr, dr, sem_s, sem_d):
        pltpu.async_copy(st_hbm.at[sidx.at[c]], sr, sem_s)
        pltpu.async_copy(dt_hbm.at[didx.at[c]], dr, sem_d)

    def process(c, sr, dr, sc, sem_s, sem_d, sem_c, first):
        pltpu.make_async_copy(st_hbm.at[sidx.at[c]], sr, sem_s).wait()
        pltpu.make_async_copy(dt_hbm.at[didx.at[c]], dr, sem_d).wait()
        @plsc.parallel_loop(jnp.int32(0), jnp.int32(CHUNK), step=np.int32(16))
        def _gate(i):
            rows = lax.iota(jnp.int32, 16) + i
            c0 = jnp.zeros((16,), jnp.int32)
            a_d = plsc.load_gather(dr, [rows, c0])
            d_d = plsc.load_gather(dr, [rows, c0 + 1])
            a_s = plsc.load_gather(sr, [rows, c0 + HIDDEN])
            d_s = plsc.load_gather(sr, [rows, c0 + HIDDEN + 1])
            t = 1.0 - 2.0 / (jnp.exp(2.0 * (a_d + a_s)) + 1.0)
            evec[pl.ds(i, 16)] = t * d_d * d_s
        @pl.when(jnp.logical_not(first))
        def _():
            pltpu.make_async_copy(sc, agg.at[didx.at[c]], sem_c).wait()
        @plsc.parallel_loop(jnp.int32(0), jnp.int32(CHUNK), step=np.int32(1), unroll=8)
        def _scale(i):
            e = plsc.load_gather(evec, [jnp.full((16,), i, jnp.int32)])
            for j in range(HIDDEN // 16):
                sc[i, pl.ds(j * 16, 16)] = sr[i, pl.ds(j * 16, 16)] * e
        pltpu.async_copy(sc, agg.at[didx.at[c]], sem_c, add=True)

    start_gather(jnp.int32(0), srA, drA, sem_sA, sem_dA)

    def _pair(p, carry):
        c0 = p * jnp.int32(2)
        start_gather(c0 + 1, srB, drB, sem_sB, sem_dB)
        process(c0, srA, drA, scA, sem_sA, sem_dA, sem_cA, p == 0)
        start_gather(c0 + 2, srA, drA, sem_sA, sem_dA)
        process(c0 + 1, srB, drB, scB, sem_sB, sem_dB, sem_cB, p == 0)
        return carry

    lax.fori_loop(jnp.int32(0), jnp.int32((NCHUNK - 1) // 2), _pair, 0,
                  unroll=False)

    process(jnp.int32(NCHUNK - 1), srA, drA, scA, sem_sA, sem_dA, sem_cA,
            jnp.bool_(False))

    pltpu.make_async_copy(scB, agg.at[didx.at[jnp.int32(0)]], sem_cB).wait()
    pltpu.make_async_copy(scA, agg.at[didx.at[jnp.int32(0)]], sem_cA).wait()

    plsc.subcore_barrier()
    pltpu.sync_copy(agg.at[pl.ds(srow0, STRIPE)],
                    out_hbm.at[cid, pl.ds(srow0, STRIPE)])


def kernel(h, adj, deg, W1, b1, Wg, bg, W2, b2):
    h = h.astype(jnp.float32)
    src = adj[0].astype(jnp.int32).reshape(NW, NCHUNK, CHUNK)
    dst = adj[1].astype(jnp.int32).reshape(NW, NCHUNK, CHUNK)
    deg2 = deg.astype(jnp.float32).reshape(N, 1)
    b1r = b1.astype(jnp.float32).reshape(1, HIDDEN)
    b2r = b2.astype(jnp.float32).reshape(1, CLASSES)
    Wg = Wg.astype(jnp.float32)
    bgr = bg.astype(jnp.float32).reshape(LAYER_NUM, 1, 1)
    zeros = jnp.zeros((STRIPE, HIDDEN), jnp.float32)

    st, dt, h0 = _encode(h, W1.astype(jnp.float32), b1r,
                         Wg[0, :HIDDEN], Wg[0, HIDDEN:], bgr[0], deg2)
    parts = None
    for l in range(LAYER_NUM):
        parts = _sc_aggregate(st, dt, src, dst, zeros)
        if l + 1 < LAYER_NUM:
            st, dt = _update(parts, h0, deg2,
                             Wg[l + 1, :HIDDEN], Wg[l + 1, HIDDEN:],
                             bgr[l + 1])
    out = _final(parts, h0, W2.astype(jnp.float32), b2r)
    return out.astype(jnp.float64)


# speedup vs baseline: 401.2632x; 1.0789x over previous
import dataclasses
import functools

import jax
import jax.numpy as jnp
import numpy as np
from jax import lax
from jax.experimental import pallas as pl
from jax.experimental.pallas import tpu as pltpu
from jax.experimental.pallas import tpu_sc as plsc

N = 10000
E = 320000
FEATURES = 128
HIDDEN = 64
CLASSES = 16
LAYER_NUM = 4
EPS = 0.3

ROWW = 48
DROWW = 16
NC = 2
NS = 16
NW = NC * NS
EPW = E // NW
CHUNK = 80
NCHUNK = EPW // CHUNK
NPAD = 10240
STRIPE = NPAD // NS


def _fill_tables(x, Wgd_ref, Wgs_ref, bg_ref, deg_ref, st_ref, dt_ref):
    a_s = jnp.dot(x, Wgs_ref[...], preferred_element_type=jnp.float32)
    a_d = jnp.dot(x, Wgd_ref[...], preferred_element_type=jnp.float32)
    a_d = a_d + bg_ref[...]
    xs = x * deg_ref[...]
    u = jax.lax.bitcast_convert_type(xs, jnp.uint32) + jnp.uint32(0x8000)
    lo = u >> jnp.uint32(16)
    hi = u & jnp.uint32(0xFFFF0000)
    wA = hi[:, 16:32] | lo[:, 0:16]
    wB = hi[:, 48:64] | lo[:, 32:48]
    words = jax.lax.bitcast_convert_type(
        jnp.concatenate([wA, wB], axis=1), jnp.float32)
    pad_s = jnp.zeros((x.shape[0], ROWW - 33), jnp.float32)
    st_ref[...] = jnp.concatenate([words, a_s, pad_s], axis=1)
    pad_d = jnp.zeros((x.shape[0], DROWW - 1), jnp.float32)
    dt_ref[...] = jnp.concatenate([a_d, pad_d], axis=1)


def _encode_body(h_ref, W1_ref, b1_ref, Wgd_ref, Wgs_ref, bg_ref, deg_ref,
                 st_ref, dt_ref, x_ref):
    x = jnp.dot(h_ref[...], W1_ref[...], preferred_element_type=jnp.float32)
    x = jnp.maximum(x + b1_ref[...], 0.0)
    _fill_tables(x, Wgd_ref, Wgs_ref, bg_ref, deg_ref, st_ref, dt_ref)
    x_ref[...] = x


def _update_body(p_ref, h0_ref, deg_ref, Wgd_ref, Wgs_ref, bg_ref,
                 st_ref, dt_ref):
    x = EPS * h0_ref[...] + deg_ref[...] * (p_ref[0, :N] + p_ref[1, :N])
    _fill_tables(x, Wgd_ref, Wgs_ref, bg_ref, deg_ref, st_ref, dt_ref)


def _final_body(p_ref, h0_ref, deg_ref, W2_ref, b2_ref, o_ref):
    x = EPS * h0_ref[...] + deg_ref[...] * (p_ref[0, :N] + p_ref[1, :N])
    logits = jnp.dot(x, W2_ref[...], preferred_element_type=jnp.float32)
    logits = logits + b2_ref[...]
    m = jnp.max(logits, axis=1, keepdims=True)
    ex = jnp.exp(logits - m)
    o_ref[...] = logits - m - jnp.log(jnp.sum(ex, axis=1, keepdims=True))


def _encode(h, W1, b1, Wgd, Wgs, bgl, deg):
    return pl.pallas_call(
        _encode_body,
        out_shape=(
            jax.ShapeDtypeStruct((N, ROWW), jnp.float32),
            jax.ShapeDtypeStruct((N, DROWW), jnp.float32),
            jax.ShapeDtypeStruct((N, HIDDEN), jnp.float32),
        ),
    )(h, W1, b1, Wgd, Wgs, bgl, deg)


def _update(parts, h0, deg, Wgd, Wgs, bgl):
    return pl.pallas_call(
        _update_body,
        out_shape=(
            jax.ShapeDtypeStruct((N, ROWW), jnp.float32),
            jax.ShapeDtypeStruct((N, DROWW), jnp.float32),
        ),
    )(parts, h0, deg, Wgd, Wgs, bgl)


def _final(parts, h0, deg, W2, b2):
    return pl.pallas_call(
        _final_body,
        out_shape=jax.ShapeDtypeStruct((N, CLASSES), jnp.float32),
    )(parts, h0, deg, W2, b2)


_MESH = plsc.VectorSubcoreMesh(core_axis_name="c", subcore_axis_name="s")

_SC_PARAMS = pltpu.CompilerParams(use_tc_tiling_on_sc=False)
if "needs_layout_passes" in pltpu.CompilerParams.__dataclass_fields__:
    _SC_PARAMS = dataclasses.replace(_SC_PARAMS, needs_layout_passes=False)


@functools.partial(
    pl.kernel,
    out_type=jax.ShapeDtypeStruct((NC, NPAD, HIDDEN), jnp.float32),
    mesh=_MESH,
    compiler_params=_SC_PARAMS,
    scratch_types=[
        pltpu.VMEM((NCHUNK, CHUNK), jnp.int32),
        pltpu.VMEM((NCHUNK, CHUNK), jnp.int32),
        pltpu.VMEM((CHUNK, ROWW), jnp.float32),
        pltpu.VMEM((CHUNK, ROWW), jnp.float32),
        pltpu.VMEM((CHUNK, DROWW), jnp.float32),
        pltpu.VMEM((CHUNK, DROWW), jnp.float32),
        pltpu.VMEM((CHUNK, HIDDEN), jnp.float32),
        pltpu.VMEM((CHUNK, HIDDEN), jnp.float32),
        pltpu.VMEM((CHUNK,), jnp.float32),
        pltpu.VMEM_SHARED((NPAD, HIDDEN), jnp.float32),
        pltpu.SemaphoreType.DMA,
        pltpu.SemaphoreType.DMA,
        pltpu.SemaphoreType.DMA,
        pltpu.SemaphoreType.DMA,
        pltpu.SemaphoreType.DMA,
        pltpu.SemaphoreType.DMA,
    ],
)
def _sc_aggregate(st_hbm, dt_hbm, src_hbm, dst_hbm, zeros_hbm, out_hbm,
                  sidx, didx, srA, srB, drA, drB, scA, scB, evec, agg,
                  sem_sA, sem_sB, sem_dA, sem_dB, sem_cA, sem_cB):
    cid = lax.axis_index("c")
    sid = lax.axis_index("s")
    wid = sid * jnp.int32(NC) + cid
    srow0 = sid * jnp.int32(STRIPE)

    pltpu.sync_copy(zeros_hbm, agg.at[pl.ds(srow0, STRIPE)])
    plsc.subcore_barrier()

    pltpu.sync_copy(src_hbm.at[wid], sidx)
    pltpu.sync_copy(dst_hbm.at[wid], didx)

    def start_gather(c, sr, dr, sem_s, sem_d):
        pltpu.async_copy(st_hbm.at[sidx.at[c]], sr, sem_s)
        pltpu.async_copy(dt_hbm.at[didx.at[c]], dr, sem_d)

    def process(c, sr, dr, sc, sem_s, sem_d, sem_c, first):
        pltpu.make_async_copy(st_hbm.at[sidx.at[c]], sr, sem_s).wait()
        pltpu.make_async_copy(dt_hbm.at[didx.at[c]], dr, sem_d).wait()
        @plsc.parallel_loop(jnp.int32(0), jnp.int32(CHUNK), step=np.int32(16))
        def _gate(i):
            rows = lax.iota(jnp.int32, 16) + i
            c0 = jnp.zeros((16,), jnp.int32)
            a_d = plsc.load_gather(dr, [rows, c0])
            a_s = plsc.load_gather(sr, [rows, c0 + 32])
            t = 1.0 - 2.0 / (jnp.exp(2.0 * (a_d + a_s)) + 1.0)
            evec[pl.ds(i, 16)] = t
        @pl.when(jnp.logical_not(first))
        def _():
            pltpu.make_async_copy(sc, agg.at[didx.at[c]], sem_c).wait()
        @plsc.parallel_loop(jnp.int32(0), jnp.int32(CHUNK), step=np.int32(1),
                            unroll=8)
        def _scale(i):
            e = plsc.load_gather(evec, [jnp.full((16,), i, jnp.int32)])
            for j in range(2):
                w = plsc.bitcast(sr[i, pl.ds(j * 16, 16)], jnp.uint32)
                xlo = plsc.bitcast(w << jnp.uint32(16), jnp.float32)
                xhi = plsc.bitcast(w & jnp.uint32(0xFFFF0000), jnp.float32)
                sc[i, pl.ds(j * 32, 16)] = xlo * e
                sc[i, pl.ds(j * 32 + 16, 16)] = xhi * e
        pltpu.async_copy(sc, agg.at[didx.at[c]], sem_c, add=True)

    start_gather(jnp.int32(0), srA, drA, sem_sA, sem_dA)

    def _pair(p, carry):
        c0 = p * jnp.int32(2)
        start_gather(c0 + 1, srB, drB, sem_sB, sem_dB)
        process(c0, srA, drA, scA, sem_sA, sem_dA, sem_cA, p == 0)
        start_gather(c0 + 2, srA, drA, sem_sA, sem_dA)
        process(c0 + 1, srB, drB, scB, sem_sB, sem_dB, sem_cB, p == 0)
        return carry

    lax.fori_loop(jnp.int32(0), jnp.int32((NCHUNK - 1) // 2), _pair, 0,
                  unroll=False)

    process(jnp.int32(NCHUNK - 1), srA, drA, scA, sem_sA, sem_dA, sem_cA,
            jnp.bool_(False))

    pltpu.make_async_copy(scB, agg.at[didx.at[jnp.int32(0)]], sem_cB).wait()
    pltpu.make_async_copy(scA, agg.at[didx.at[jnp.int32(0)]], sem_cA).wait()

    plsc.subcore_barrier()
    pltpu.sync_copy(agg.at[pl.ds(srow0, STRIPE)],
                    out_hbm.at[cid, pl.ds(srow0, STRIPE)])


def kernel(h, adj, deg, W1, b1, Wg, bg, W2, b2):
    h = h.astype(jnp.float32)
    src = adj[0].astype(jnp.int32).reshape(NW, NCHUNK, CHUNK)
    dst = adj[1].astype(jnp.int32).reshape(NW, NCHUNK, CHUNK)
    deg2 = deg.astype(jnp.float32).reshape(N, 1)
    b1r = b1.astype(jnp.float32).reshape(1, HIDDEN)
    b2r = b2.astype(jnp.float32).reshape(1, CLASSES)
    Wg = Wg.astype(jnp.float32)
    bgr = bg.astype(jnp.float32).reshape(LAYER_NUM, 1, 1)
    zeros = jnp.zeros((STRIPE, HIDDEN), jnp.float32)

    st, dt, h0 = _encode(h, W1.astype(jnp.float32), b1r,
                         Wg[0, :HIDDEN], Wg[0, HIDDEN:], bgr[0], deg2)
    parts = None
    for l in range(LAYER_NUM):
        parts = _sc_aggregate(st, dt, src, dst, zeros)
        if l + 1 < LAYER_NUM:
            st, dt = _update(parts, h0, deg2,
                             Wg[l + 1, :HIDDEN], Wg[l + 1, HIDDEN:],
                             bgr[l + 1])
    out = _final(parts, h0, deg2, W2.astype(jnp.float32), b2r)
    return out.astype(jnp.float64)
